# Initial kernel scaffold; baseline (speedup 1.0000x reference)
import jax
import jax.numpy as jnp
from jax.experimental import pallas as pl


def kernel(x, table):
    raise NotImplementedError("write your pallas kernel here")


# speedup vs baseline: 13.5959x; 13.5959x over previous
import functools

import jax
import jax.numpy as jnp
from jax import lax
from jax.experimental import pallas as pl
from jax.experimental.pallas import tpu as pltpu
from jax.experimental.pallas import tpu_sc as plsc

EMBED_DIM = 4096
BATCH = 1024

NC = 2
NS = 16
NW = NC * NS
B_PER_W = BATCH // NW
CHUNK = 8
NCHUNK = B_PER_W // CHUNK

_mesh = plsc.VectorSubcoreMesh(core_axis_name="c", subcore_axis_name="s")


@functools.partial(
    pl.kernel,
    mesh=_mesh,
    out_type=jax.ShapeDtypeStruct((BATCH, EMBED_DIM), jnp.float32),
    scratch_types=[
        pltpu.VMEM((NCHUNK, CHUNK), jnp.int32),
        pltpu.VMEM((CHUNK, EMBED_DIM), jnp.float32),
        pltpu.VMEM((CHUNK, EMBED_DIM), jnp.float32),
        pltpu.SemaphoreType.DMA,
        pltpu.SemaphoreType.DMA,
        pltpu.SemaphoreType.DMA,
        pltpu.SemaphoreType.DMA,
    ],
)
def _cls_gather(idx_hbm, table_hbm, out_hbm, idx_v, buf0, buf1,
                sg0, sg1, sw0, sw1):
    wid = lax.axis_index("s") * NC + lax.axis_index("c")
    base = wid * B_PER_W
    pltpu.sync_copy(idx_hbm.at[pl.ds(wid * NCHUNK, NCHUNK)], idx_v)
    g0 = pltpu.async_copy(table_hbm.at[idx_v.at[0]], buf0, sg0)
    g1 = pltpu.async_copy(table_hbm.at[idx_v.at[1]], buf1, sg1)
    g0.wait()
    w0 = pltpu.async_copy(buf0, out_hbm.at[pl.ds(base, CHUNK)], sw0)
    g1.wait()
    w1 = pltpu.async_copy(buf1, out_hbm.at[pl.ds(base + CHUNK, CHUNK)], sw1)
    w0.wait()
    g2 = pltpu.async_copy(table_hbm.at[idx_v.at[2]], buf0, sg0)
    w1.wait()
    g3 = pltpu.async_copy(table_hbm.at[idx_v.at[3]], buf1, sg1)
    g2.wait()
    w2 = pltpu.async_copy(buf0, out_hbm.at[pl.ds(base + 2 * CHUNK, CHUNK)], sw0)
    g3.wait()
    w3 = pltpu.async_copy(buf1, out_hbm.at[pl.ds(base + 3 * CHUNK, CHUNK)], sw1)
    w2.wait()
    w3.wait()


def kernel(x, table):
    idx = x[:, 0].reshape(BATCH // CHUNK, CHUNK)
    out = _cls_gather(idx, table)
    return (out, out)
